# Initial kernel scaffold; baseline (speedup 1.0000x reference)
import jax
import jax.numpy as jnp
from jax.experimental import pallas as pl


def kernel(x, edge_index, edge_attr, W1, b1, W2, b2):
    raise NotImplementedError("write your pallas kernel here")


# speedup vs baseline: 13.0229x; 13.0229x over previous
import functools

import jax
import jax.numpy as jnp
from jax import lax
from jax.experimental import pallas as pl
from jax.experimental.pallas import tpu as pltpu
from jax.experimental.pallas import tpu_sc as plsc

N_NODES = 10000
N_PAD = 10240
E_EDGES = 320000
NW = 32
EPW = 10240
E_PAD = NW * EPW
CHUNK = 128
CPW = EPW // CHUNK
D = 16

_mesh = plsc.VectorSubcoreMesh(core_axis_name="c", subcore_axis_name="s")


@functools.partial(
    pl.kernel,
    mesh=_mesh,
    out_type=jax.ShapeDtypeStruct((2, N_PAD), jnp.float32),
    scratch_types=[
        pltpu.VMEM((CHUNK,), jnp.int32),
        pltpu.VMEM((CHUNK,), jnp.float32),
        pltpu.VMEM_SHARED((N_PAD,), jnp.float32),
        pltpu.SemaphoreType.DMA,
    ],
    compiler_params=pltpu.CompilerParams(use_tc_tiling_on_sc=False),
)
def _sc_degree(col_hbm, w_hbm, zero_hbm, out_hbm, colbuf, wbuf, deg_sp, sem):
    c = lax.axis_index("c")
    s = lax.axis_index("s")
    wid = s * 2 + c
    base = wid * EPW

    @pl.when(s == 0)
    def _():
        pltpu.sync_copy(zero_hbm, deg_sp)

    plsc.subcore_barrier()

    def body(k, carry):
        off = base + k * CHUNK
        pltpu.sync_copy(col_hbm.at[pl.ds(off, CHUNK)], colbuf)
        pltpu.sync_copy(w_hbm.at[pl.ds(off, CHUNK)], wbuf)
        pltpu.sync_copy(wbuf, deg_sp.at[colbuf], add=True)
        return carry

    lax.fori_loop(0, CPW, body, 0)

    plsc.subcore_barrier()
    rows = N_PAD // 16
    pltpu.sync_copy(
        deg_sp.at[pl.ds(s * rows, rows)],
        out_hbm.at[c, pl.ds(s * rows, rows)],
    )


@functools.partial(
    pl.kernel,
    mesh=_mesh,
    out_type=jax.ShapeDtypeStruct((2, N_PAD, D), jnp.float32),
    scratch_types=[
        pltpu.VMEM((CHUNK,), jnp.int32),
        pltpu.VMEM((CHUNK,), jnp.int32),
        pltpu.VMEM((CHUNK,), jnp.float32),
        pltpu.VMEM((CHUNK, D), jnp.float32),
        pltpu.VMEM_SHARED((N_PAD, D), jnp.float32),
        pltpu.SemaphoreType.DMA,
    ],
    compiler_params=pltpu.CompilerParams(use_tc_tiling_on_sc=False),
)
def _sc_edge(y_hbm, row_hbm, col_hbm, w_hbm, zero_hbm, out_hbm,
             rowbuf, colbuf, wbuf, msgbuf, agg_sp, sem):
    c = lax.axis_index("c")
    s = lax.axis_index("s")
    wid = s * 2 + c
    base = wid * EPW

    @pl.when(s == 0)
    def _():
        pltpu.sync_copy(zero_hbm, agg_sp)

    plsc.subcore_barrier()

    def body(k, carry):
        off = base + k * CHUNK
        pltpu.sync_copy(row_hbm.at[pl.ds(off, CHUNK)], rowbuf)
        pltpu.sync_copy(col_hbm.at[pl.ds(off, CHUNK)], colbuf)
        pltpu.sync_copy(w_hbm.at[pl.ds(off, CHUNK)], wbuf)
        pltpu.async_copy(y_hbm.at[rowbuf], msgbuf, sem).wait()
        for g in range(CHUNK // 16):
            wv = wbuf[pl.ds(g * 16, 16)]
            for j in range(16):
                e = g * 16 + j
                msgbuf[e, :] = msgbuf[e, :] * wv[j]
        pltpu.sync_copy(msgbuf, agg_sp.at[colbuf], add=True)
        return carry

    lax.fori_loop(0, CPW, body, 0)

    plsc.subcore_barrier()
    rows = N_PAD // 16
    pltpu.sync_copy(
        agg_sp.at[pl.ds(s * rows, rows)],
        out_hbm.at[c, pl.ds(s * rows, rows)],
    )


def _tc1_body(pdeg_ref, x_ref, w1_ref, dis_ref, y_ref):
    deg = pdeg_ref[0] + pdeg_ref[1]
    dis = jnp.where(deg > 0.0, lax.rsqrt(jnp.where(deg > 0.0, deg, 1.0)), 0.0)
    dis_ref[...] = dis
    xw = jnp.dot(x_ref[...], w1_ref[...], preferred_element_type=jnp.float32)
    y_ref[...] = xw * dis


_tc1 = pl.pallas_call(
    _tc1_body,
    out_shape=[
        jax.ShapeDtypeStruct((N_PAD, 1), jnp.float32),
        jax.ShapeDtypeStruct((N_PAD, D), jnp.float32),
    ],
)


def _tc2_body(p_ref, dis_ref, b1_ref, w2_ref, y2_ref):
    dis = dis_ref[...]
    h = jnp.maximum((p_ref[0] + p_ref[1]) * dis + b1_ref[...], 0.0)
    xw2 = jnp.dot(h, w2_ref[...], preferred_element_type=jnp.float32)
    y2_ref[...] = xw2 * dis


_tc2 = pl.pallas_call(
    _tc2_body,
    out_shape=jax.ShapeDtypeStruct((N_PAD, D), jnp.float32),
)


def _tc3_body(q_ref, dis_ref, b2_ref, out_ref):
    out_ref[...] = jnp.maximum(
        (q_ref[0] + q_ref[1]) * dis_ref[...] + b2_ref[...], 0.0
    )


_tc3 = pl.pallas_call(
    _tc3_body,
    out_shape=jax.ShapeDtypeStruct((N_PAD, D), jnp.float32),
)


@jax.jit
def kernel(x, edge_index, edge_attr, W1, b1, W2, b2):
    row = edge_index[0]
    col = edge_index[1]
    pad_e = E_PAD - E_EDGES
    row_p = jnp.concatenate([row, jnp.zeros((pad_e,), jnp.int32)])
    col_p = jnp.concatenate([col, jnp.zeros((pad_e,), jnp.int32)])
    w_p = jnp.concatenate([edge_attr, jnp.zeros((pad_e,), jnp.float32)])

    x_p = jnp.concatenate(
        [x, jnp.zeros((N_PAD - N_NODES, x.shape[1]), jnp.float32)]
    )
    zero1 = jnp.zeros((N_PAD,), jnp.float32)
    zero2 = jnp.zeros((N_PAD, D), jnp.float32)

    pdeg = _sc_degree(col_p, w_p, zero1)
    dis, y1 = _tc1(pdeg.reshape(2, N_PAD, 1), x_p, W1)

    p1 = _sc_edge(y1, row_p, col_p, w_p, zero2)
    y2 = _tc2(p1, dis, b1.reshape(1, D), W2)

    p2 = _sc_edge(y2, row_p, col_p, w_p, zero2)
    out = _tc3(p2, dis, b2.reshape(1, D))
    return out[:N_NODES]


# speedup vs baseline: 28.1834x; 2.1641x over previous
import functools

import jax
import jax.numpy as jnp
from jax import lax
from jax.experimental import pallas as pl
from jax.experimental.pallas import tpu as pltpu
from jax.experimental.pallas import tpu_sc as plsc

N_NODES = 10000
N_PAD = 10240
E_EDGES = 320000
NW = 32
EPW = 10240
E_PAD = NW * EPW
CHUNK = 128
CPW = EPW // CHUNK
D = 16

_mesh = plsc.VectorSubcoreMesh(core_axis_name="c", subcore_axis_name="s")


@functools.partial(
    pl.kernel,
    mesh=_mesh,
    out_type=jax.ShapeDtypeStruct((2, N_PAD), jnp.float32),
    scratch_types=[
        pltpu.VMEM((CPW, CHUNK), jnp.int32),
        pltpu.VMEM((CPW, CHUNK), jnp.float32),
        pltpu.VMEM_SHARED((N_PAD,), jnp.float32),
        pltpu.SemaphoreType.DMA,
        pltpu.SemaphoreType.DMA,
    ],
    compiler_params=pltpu.CompilerParams(use_tc_tiling_on_sc=False),
)
def _sc_degree(col_hbm, w_hbm, zero_hbm, out_hbm, colbuf, wbuf, deg_sp,
               stage_sem, ssem):
    c = lax.axis_index("c")
    s = lax.axis_index("s")
    wid = s * 2 + c

    @pl.when(s == 0)
    def _():
        pltpu.sync_copy(zero_hbm, deg_sp)

    pltpu.async_copy(col_hbm.at[pl.ds(wid * CPW, CPW)], colbuf, stage_sem)
    pltpu.async_copy(w_hbm.at[pl.ds(wid * CPW, CPW)], wbuf, stage_sem)
    pltpu.make_async_copy(col_hbm.at[pl.ds(wid * CPW, CPW)], colbuf,
                          stage_sem).wait()
    pltpu.make_async_copy(w_hbm.at[pl.ds(wid * CPW, CPW)], wbuf,
                          stage_sem).wait()

    plsc.subcore_barrier()

    def body(k, carry):
        pltpu.async_copy(wbuf.at[k], deg_sp.at[colbuf.at[k]], ssem, add=True)
        return carry

    lax.fori_loop(0, CPW, body, 0)

    def drain(k, carry):
        pltpu.make_async_copy(wbuf.at[k], deg_sp.at[colbuf.at[k]], ssem).wait()
        return carry

    lax.fori_loop(0, CPW, drain, 0)

    plsc.subcore_barrier()
    rows = N_PAD // 16
    pltpu.sync_copy(
        deg_sp.at[pl.ds(s * rows, rows)],
        out_hbm.at[c, pl.ds(s * rows, rows)],
    )


@functools.partial(
    pl.kernel,
    mesh=_mesh,
    out_type=jax.ShapeDtypeStruct((2, N_PAD, D), jnp.float32),
    scratch_types=[
        pltpu.VMEM((CPW, CHUNK), jnp.int32),
        pltpu.VMEM((CPW, CHUNK), jnp.int32),
        pltpu.VMEM((CPW, CHUNK), jnp.float32),
        pltpu.VMEM((2, CHUNK, D), jnp.float32),
        pltpu.VMEM_SHARED((N_PAD, D), jnp.float32),
        pltpu.SemaphoreType.DMA,
        pltpu.SemaphoreType.DMA,
    ],
    compiler_params=pltpu.CompilerParams(use_tc_tiling_on_sc=False),
)
def _sc_edge(y_hbm, row_hbm, col_hbm, w_hbm, zero_hbm, out_hbm,
             rowbuf, colbuf, wbuf, msgbuf, agg_sp, stage_sem, gsem):
    c = lax.axis_index("c")
    s = lax.axis_index("s")
    wid = s * 2 + c

    @pl.when(s == 0)
    def _():
        pltpu.sync_copy(zero_hbm, agg_sp)

    rows_slice = pl.ds(wid * CPW, CPW)
    pltpu.async_copy(row_hbm.at[rows_slice], rowbuf, stage_sem)
    pltpu.async_copy(col_hbm.at[rows_slice], colbuf, stage_sem)
    pltpu.async_copy(w_hbm.at[rows_slice], wbuf, stage_sem)
    pltpu.make_async_copy(row_hbm.at[rows_slice], rowbuf, stage_sem).wait()
    pltpu.make_async_copy(col_hbm.at[rows_slice], colbuf, stage_sem).wait()
    pltpu.make_async_copy(w_hbm.at[rows_slice], wbuf, stage_sem).wait()

    plsc.subcore_barrier()

    pltpu.async_copy(y_hbm.at[rowbuf.at[0]], msgbuf.at[0], gsem)

    def body(k, carry):
        slot = lax.rem(k, 2)
        nxt = lax.rem(k + 1, 2)

        @pl.when(k + 1 < CPW)
        def _():
            pltpu.async_copy(y_hbm.at[rowbuf.at[k + 1]], msgbuf.at[nxt], gsem)

        pltpu.make_async_copy(y_hbm.at[rowbuf.at[k]], msgbuf.at[slot],
                              gsem).wait()
        for g in range(CHUNK // 16):
            wv = wbuf[k, pl.ds(g * 16, 16)]
            for j in range(16):
                e = g * 16 + j
                msgbuf[slot, e, :] = msgbuf[slot, e, :] * wv[j]
        pltpu.sync_copy(msgbuf.at[slot], agg_sp.at[colbuf.at[k]], add=True)
        return carry

    lax.fori_loop(0, CPW, body, 0)

    plsc.subcore_barrier()
    rows = N_PAD // 16
    pltpu.sync_copy(
        agg_sp.at[pl.ds(s * rows, rows)],
        out_hbm.at[c, pl.ds(s * rows, rows)],
    )


def _tc1_body(pdeg_ref, x_ref, w1_ref, dis_ref, y_ref):
    deg = pdeg_ref[0] + pdeg_ref[1]
    dis = jnp.where(deg > 0.0, lax.rsqrt(jnp.where(deg > 0.0, deg, 1.0)), 0.0)
    dis_ref[...] = dis
    xw = jnp.dot(x_ref[...], w1_ref[...], preferred_element_type=jnp.float32)
    y_ref[...] = xw * dis


_tc1 = pl.pallas_call(
    _tc1_body,
    out_shape=[
        jax.ShapeDtypeStruct((N_PAD, 1), jnp.float32),
        jax.ShapeDtypeStruct((N_PAD, D), jnp.float32),
    ],
)


def _tc2_body(p_ref, dis_ref, b1_ref, w2_ref, y2_ref):
    dis = dis_ref[...]
    h = jnp.maximum((p_ref[0] + p_ref[1]) * dis + b1_ref[...], 0.0)
    xw2 = jnp.dot(h, w2_ref[...], preferred_element_type=jnp.float32)
    y2_ref[...] = xw2 * dis


_tc2 = pl.pallas_call(
    _tc2_body,
    out_shape=jax.ShapeDtypeStruct((N_PAD, D), jnp.float32),
)


def _tc3_body(q_ref, dis_ref, b2_ref, out_ref):
    out_ref[...] = jnp.maximum(
        (q_ref[0] + q_ref[1]) * dis_ref[...] + b2_ref[...], 0.0
    )


_tc3 = pl.pallas_call(
    _tc3_body,
    out_shape=jax.ShapeDtypeStruct((N_PAD, D), jnp.float32),
)


@jax.jit
def kernel(x, edge_index, edge_attr, W1, b1, W2, b2):
    row = edge_index[0]
    col = edge_index[1]
    pad_e = E_PAD - E_EDGES
    row_p = jnp.concatenate([row, jnp.zeros((pad_e,), jnp.int32)])
    col_p = jnp.concatenate([col, jnp.zeros((pad_e,), jnp.int32)])
    w_p = jnp.concatenate([edge_attr, jnp.zeros((pad_e,), jnp.float32)])
    row_p = row_p.reshape(NW * CPW, CHUNK)
    col_p = col_p.reshape(NW * CPW, CHUNK)
    w_p = w_p.reshape(NW * CPW, CHUNK)

    x_p = jnp.concatenate(
        [x, jnp.zeros((N_PAD - N_NODES, x.shape[1]), jnp.float32)]
    )
    zero1 = jnp.zeros((N_PAD,), jnp.float32)
    zero2 = jnp.zeros((N_PAD, D), jnp.float32)

    pdeg = _sc_degree(col_p, w_p, zero1)
    dis, y1 = _tc1(pdeg.reshape(2, N_PAD, 1), x_p, W1)

    p1 = _sc_edge(y1, row_p, col_p, w_p, zero2)
    y2 = _tc2(p1, dis, b1.reshape(1, D), W2)

    p2 = _sc_edge(y2, row_p, col_p, w_p, zero2)
    out = _tc3(p2, dis, b2.reshape(1, D))
    return out[:N_NODES]


# speedup vs baseline: 31.3559x; 1.1126x over previous
import functools

import jax
import jax.numpy as jnp
from jax import lax
from jax.experimental import pallas as pl
from jax.experimental.pallas import tpu as pltpu
from jax.experimental.pallas import tpu_sc as plsc

N_NODES = 10000
N_PAD = 10240
E_EDGES = 320000
NW = 32
EPW = 10240
E_PAD = NW * EPW
CHUNK = 128
CPW = EPW // CHUNK
D = 16

_mesh = plsc.VectorSubcoreMesh(core_axis_name="c", subcore_axis_name="s")


@functools.partial(
    pl.kernel,
    mesh=_mesh,
    out_type=jax.ShapeDtypeStruct((2, N_PAD), jnp.float32),
    scratch_types=[
        pltpu.VMEM((CPW, CHUNK), jnp.int32),
        pltpu.VMEM((CPW, CHUNK), jnp.float32),
        pltpu.VMEM_SHARED((N_PAD,), jnp.float32),
        pltpu.SemaphoreType.DMA,
        pltpu.SemaphoreType.DMA,
    ],
    compiler_params=pltpu.CompilerParams(use_tc_tiling_on_sc=False),
)
def _sc_degree(col_hbm, w_hbm, zero_hbm, out_hbm, colbuf, wbuf, deg_sp,
               stage_sem, ssem):
    c = lax.axis_index("c")
    s = lax.axis_index("s")
    wid = s * 2 + c

    @pl.when(s == 0)
    def _():
        pltpu.sync_copy(zero_hbm, deg_sp)

    pltpu.async_copy(col_hbm.at[pl.ds(wid * CPW, CPW)], colbuf, stage_sem)
    pltpu.async_copy(w_hbm.at[pl.ds(wid * CPW, CPW)], wbuf, stage_sem)
    pltpu.make_async_copy(col_hbm.at[pl.ds(wid * CPW, CPW)], colbuf,
                          stage_sem).wait()
    pltpu.make_async_copy(w_hbm.at[pl.ds(wid * CPW, CPW)], wbuf,
                          stage_sem).wait()

    plsc.subcore_barrier()

    def body(k, carry):
        pltpu.async_copy(wbuf.at[k], deg_sp.at[colbuf.at[k]], ssem, add=True)
        return carry

    lax.fori_loop(0, CPW, body, 0)

    def drain(k, carry):
        pltpu.make_async_copy(wbuf.at[k], deg_sp.at[colbuf.at[k]], ssem).wait()
        return carry

    lax.fori_loop(0, CPW, drain, 0)

    plsc.subcore_barrier()
    rows = N_PAD // 16
    pltpu.sync_copy(
        deg_sp.at[pl.ds(s * rows, rows)],
        out_hbm.at[c, pl.ds(s * rows, rows)],
    )


@functools.partial(
    pl.kernel,
    mesh=_mesh,
    out_type=jax.ShapeDtypeStruct((2, N_PAD, D), jnp.float32),
    scratch_types=[
        pltpu.VMEM((CPW, CHUNK), jnp.int32),
        pltpu.VMEM((CPW, CHUNK), jnp.int32),
        pltpu.VMEM((CPW, CHUNK), jnp.float32),
        pltpu.VMEM((4, CHUNK, D), jnp.float32),
        pltpu.VMEM_SHARED((N_PAD, D), jnp.float32),
        pltpu.SemaphoreType.DMA,
        pltpu.SemaphoreType.DMA,
        pltpu.SemaphoreType.DMA,
    ],
    compiler_params=pltpu.CompilerParams(use_tc_tiling_on_sc=False),
)
def _sc_edge(y_hbm, row_hbm, col_hbm, w_hbm, zero_hbm, out_hbm,
             rowbuf, colbuf, wbuf, msgbuf, agg_sp, stage_sem, gsem, ssem):
    c = lax.axis_index("c")
    s = lax.axis_index("s")
    wid = s * 2 + c

    @pl.when(s == 0)
    def _():
        pltpu.sync_copy(zero_hbm, agg_sp)

    rows_slice = pl.ds(wid * CPW, CPW)
    pltpu.async_copy(row_hbm.at[rows_slice], rowbuf, stage_sem)
    pltpu.async_copy(col_hbm.at[rows_slice], colbuf, stage_sem)
    pltpu.async_copy(w_hbm.at[rows_slice], wbuf, stage_sem)
    pltpu.make_async_copy(row_hbm.at[rows_slice], rowbuf, stage_sem).wait()
    pltpu.make_async_copy(col_hbm.at[rows_slice], colbuf, stage_sem).wait()
    pltpu.make_async_copy(w_hbm.at[rows_slice], wbuf, stage_sem).wait()

    plsc.subcore_barrier()

    pltpu.async_copy(y_hbm.at[rowbuf.at[0]], msgbuf.at[0], gsem)
    pltpu.async_copy(y_hbm.at[rowbuf.at[1]], msgbuf.at[1], gsem)

    def body(k, carry):
        slot = lax.rem(k, 4)

        @pl.when(k >= 2)
        def _():
            km2 = k - 2
            pltpu.make_async_copy(msgbuf.at[lax.rem(km2, 4)],
                                  agg_sp.at[colbuf.at[km2]], ssem).wait()

        @pl.when(k + 2 < CPW)
        def _():
            pltpu.async_copy(y_hbm.at[rowbuf.at[k + 2]],
                             msgbuf.at[lax.rem(k + 2, 4)], gsem)

        pltpu.make_async_copy(y_hbm.at[rowbuf.at[k]], msgbuf.at[slot],
                              gsem).wait()
        for g in range(CHUNK // 16):
            wv = wbuf[k, pl.ds(g * 16, 16)]
            for j in range(16):
                e = g * 16 + j
                msgbuf[slot, e, :] = msgbuf[slot, e, :] * wv[j]
        pltpu.async_copy(msgbuf.at[slot], agg_sp.at[colbuf.at[k]], ssem,
                         add=True)
        return carry

    lax.fori_loop(0, CPW, body, 0)

    def drain(k, carry):
        pltpu.make_async_copy(msgbuf.at[lax.rem(k, 4)],
                              agg_sp.at[colbuf.at[k]], ssem).wait()
        return carry

    lax.fori_loop(CPW - 2, CPW, drain, 0)

    plsc.subcore_barrier()
    rows = N_PAD // 16
    pltpu.sync_copy(
        agg_sp.at[pl.ds(s * rows, rows)],
        out_hbm.at[c, pl.ds(s * rows, rows)],
    )


def _tc1_body(pdeg_ref, x_ref, w1_ref, dis_ref, y_ref):
    deg = pdeg_ref[0] + pdeg_ref[1]
    dis = jnp.where(deg > 0.0, lax.rsqrt(jnp.where(deg > 0.0, deg, 1.0)), 0.0)
    dis_ref[...] = dis
    xw = jnp.dot(x_ref[...], w1_ref[...], preferred_element_type=jnp.float32)
    y_ref[...] = xw * dis


_tc1 = pl.pallas_call(
    _tc1_body,
    out_shape=[
        jax.ShapeDtypeStruct((N_PAD, 1), jnp.float32),
        jax.ShapeDtypeStruct((N_PAD, D), jnp.float32),
    ],
)


def _tc2_body(p_ref, dis_ref, b1_ref, w2_ref, y2_ref):
    dis = dis_ref[...]
    h = jnp.maximum((p_ref[0] + p_ref[1]) * dis + b1_ref[...], 0.0)
    xw2 = jnp.dot(h, w2_ref[...], preferred_element_type=jnp.float32)
    y2_ref[...] = xw2 * dis


_tc2 = pl.pallas_call(
    _tc2_body,
    out_shape=jax.ShapeDtypeStruct((N_PAD, D), jnp.float32),
)


def _tc3_body(q_ref, dis_ref, b2_ref, out_ref):
    out_ref[...] = jnp.maximum(
        (q_ref[0] + q_ref[1]) * dis_ref[...] + b2_ref[...], 0.0
    )


_tc3 = pl.pallas_call(
    _tc3_body,
    out_shape=jax.ShapeDtypeStruct((N_PAD, D), jnp.float32),
)


@jax.jit
def kernel(x, edge_index, edge_attr, W1, b1, W2, b2):
    row = edge_index[0]
    col = edge_index[1]
    pad_e = E_PAD - E_EDGES
    row_p = jnp.concatenate([row, jnp.zeros((pad_e,), jnp.int32)])
    col_p = jnp.concatenate([col, jnp.zeros((pad_e,), jnp.int32)])
    w_p = jnp.concatenate([edge_attr, jnp.zeros((pad_e,), jnp.float32)])
    row_p = row_p.reshape(NW * CPW, CHUNK)
    col_p = col_p.reshape(NW * CPW, CHUNK)
    w_p = w_p.reshape(NW * CPW, CHUNK)

    x_p = jnp.concatenate(
        [x, jnp.zeros((N_PAD - N_NODES, x.shape[1]), jnp.float32)]
    )
    zero1 = jnp.zeros((N_PAD,), jnp.float32)
    zero2 = jnp.zeros((N_PAD, D), jnp.float32)

    pdeg = _sc_degree(col_p, w_p, zero1)
    dis, y1 = _tc1(pdeg.reshape(2, N_PAD, 1), x_p, W1)

    p1 = _sc_edge(y1, row_p, col_p, w_p, zero2)
    y2 = _tc2(p1, dis, b1.reshape(1, D), W2)

    p2 = _sc_edge(y2, row_p, col_p, w_p, zero2)
    out = _tc3(p2, dis, b2.reshape(1, D))
    return out[:N_NODES]


# speedup vs baseline: 46.3180x; 1.4772x over previous
import functools

import jax
import jax.numpy as jnp
from jax import lax
from jax.experimental import pallas as pl
from jax.experimental.pallas import tpu as pltpu
from jax.experimental.pallas import tpu_sc as plsc

N_NODES = 10000
N_PAD = 10240
E_EDGES = 320000
NW = 32
EPW = 10240
E_PAD = NW * EPW
CHUNK = 128
CPW = EPW // CHUNK
D = 16

_mesh = plsc.VectorSubcoreMesh(core_axis_name="c", subcore_axis_name="s")


@functools.partial(
    pl.kernel,
    mesh=_mesh,
    out_type=jax.ShapeDtypeStruct((2, N_PAD), jnp.float32),
    scratch_types=[
        pltpu.VMEM((CPW, CHUNK), jnp.int32),
        pltpu.VMEM((CPW, CHUNK), jnp.float32),
        pltpu.VMEM_SHARED((N_PAD,), jnp.float32),
        pltpu.SemaphoreType.DMA,
        pltpu.SemaphoreType.DMA,
    ],
    compiler_params=pltpu.CompilerParams(use_tc_tiling_on_sc=False),
)
def _sc_degree(col_hbm, w_hbm, zero_hbm, out_hbm, colbuf, wbuf, deg_sp,
               stage_sem, ssem):
    c = lax.axis_index("c")
    s = lax.axis_index("s")
    wid = s * 2 + c

    @pl.when(s == 0)
    def _():
        pltpu.sync_copy(zero_hbm, deg_sp)

    pltpu.async_copy(col_hbm.at[pl.ds(wid * CPW, CPW)], colbuf, stage_sem)
    pltpu.async_copy(w_hbm.at[pl.ds(wid * CPW, CPW)], wbuf, stage_sem)
    pltpu.make_async_copy(col_hbm.at[pl.ds(wid * CPW, CPW)], colbuf,
                          stage_sem).wait()
    pltpu.make_async_copy(w_hbm.at[pl.ds(wid * CPW, CPW)], wbuf,
                          stage_sem).wait()

    plsc.subcore_barrier()

    def body(k, carry):
        pltpu.async_copy(wbuf.at[k], deg_sp.at[colbuf.at[k]], ssem, add=True)
        return carry

    lax.fori_loop(0, CPW, body, 0)

    def drain(k, carry):
        pltpu.make_async_copy(wbuf.at[k], deg_sp.at[colbuf.at[k]], ssem).wait()
        return carry

    lax.fori_loop(0, CPW, drain, 0)

    plsc.subcore_barrier()
    rows = N_PAD // 16
    pltpu.sync_copy(
        deg_sp.at[pl.ds(s * rows, rows)],
        out_hbm.at[c, pl.ds(s * rows, rows)],
    )


@functools.partial(
    pl.kernel,
    mesh=_mesh,
    out_type=jax.ShapeDtypeStruct((2, N_PAD, D), jnp.float32),
    scratch_types=[
        pltpu.VMEM((CPW, CHUNK), jnp.int32),
        pltpu.VMEM((CPW, CHUNK), jnp.int32),
        pltpu.VMEM((CPW, CHUNK), jnp.float32),
        pltpu.VMEM((4, CHUNK, D), jnp.float32),
        pltpu.VMEM_SHARED((N_PAD, D), jnp.float32),
        pltpu.VMEM_SHARED((N_PAD, D), jnp.float32),
        pltpu.SemaphoreType.DMA,
        pltpu.SemaphoreType.DMA,
        pltpu.SemaphoreType.DMA,
    ],
    compiler_params=pltpu.CompilerParams(use_tc_tiling_on_sc=False),
)
def _sc_edge(y_hbm, row_hbm, col_hbm, w_hbm, zero_hbm, out_hbm,
             rowbuf, colbuf, wbuf, msgbuf, agg_sp, y_sp,
             stage_sem, gsem, ssem):
    c = lax.axis_index("c")
    s = lax.axis_index("s")
    wid = s * 2 + c

    @pl.when(s == 0)
    def _():
        pltpu.sync_copy(zero_hbm, agg_sp)

    nrows = N_PAD // 16
    node_slice = pl.ds(s * nrows, nrows)
    pltpu.sync_copy(y_hbm.at[node_slice], y_sp.at[node_slice])

    rows_slice = pl.ds(wid * CPW, CPW)
    pltpu.async_copy(row_hbm.at[rows_slice], rowbuf, stage_sem)
    pltpu.async_copy(col_hbm.at[rows_slice], colbuf, stage_sem)
    pltpu.async_copy(w_hbm.at[rows_slice], wbuf, stage_sem)
    pltpu.make_async_copy(row_hbm.at[rows_slice], rowbuf, stage_sem).wait()
    pltpu.make_async_copy(col_hbm.at[rows_slice], colbuf, stage_sem).wait()
    pltpu.make_async_copy(w_hbm.at[rows_slice], wbuf, stage_sem).wait()

    plsc.subcore_barrier()

    pltpu.async_copy(y_sp.at[rowbuf.at[0]], msgbuf.at[0], gsem)
    pltpu.async_copy(y_sp.at[rowbuf.at[1]], msgbuf.at[1], gsem)

    def body(k, carry):
        slot = lax.rem(k, 4)

        @pl.when(k >= 2)
        def _():
            km2 = k - 2
            pltpu.make_async_copy(msgbuf.at[lax.rem(km2, 4)],
                                  agg_sp.at[colbuf.at[km2]], ssem).wait()

        @pl.when(k + 2 < CPW)
        def _():
            pltpu.async_copy(y_sp.at[rowbuf.at[k + 2]],
                             msgbuf.at[lax.rem(k + 2, 4)], gsem)

        pltpu.make_async_copy(y_sp.at[rowbuf.at[k]], msgbuf.at[slot],
                              gsem).wait()
        for g in range(CHUNK // 16):
            wv = wbuf[k, pl.ds(g * 16, 16)]
            for j in range(16):
                e = g * 16 + j
                msgbuf[slot, e, :] = msgbuf[slot, e, :] * wv[j]
        pltpu.async_copy(msgbuf.at[slot], agg_sp.at[colbuf.at[k]], ssem,
                         add=True)
        return carry

    lax.fori_loop(0, CPW, body, 0)

    def drain(k, carry):
        pltpu.make_async_copy(msgbuf.at[lax.rem(k, 4)],
                              agg_sp.at[colbuf.at[k]], ssem).wait()
        return carry

    lax.fori_loop(CPW - 2, CPW, drain, 0)

    plsc.subcore_barrier()
    rows = N_PAD // 16
    pltpu.sync_copy(
        agg_sp.at[pl.ds(s * rows, rows)],
        out_hbm.at[c, pl.ds(s * rows, rows)],
    )


def _tc1_body(pdeg_ref, x_ref, w1_ref, dis_ref, y_ref):
    deg = pdeg_ref[0] + pdeg_ref[1]
    dis = jnp.where(deg > 0.0, lax.rsqrt(jnp.where(deg > 0.0, deg, 1.0)), 0.0)
    dis_ref[...] = dis
    xw = jnp.dot(x_ref[...], w1_ref[...], preferred_element_type=jnp.float32)
    y_ref[...] = xw * dis


_tc1 = pl.pallas_call(
    _tc1_body,
    out_shape=[
        jax.ShapeDtypeStruct((N_PAD, 1), jnp.float32),
        jax.ShapeDtypeStruct((N_PAD, D), jnp.float32),
    ],
)


def _tc2_body(p_ref, dis_ref, b1_ref, w2_ref, y2_ref):
    dis = dis_ref[...]
    h = jnp.maximum((p_ref[0] + p_ref[1]) * dis + b1_ref[...], 0.0)
    xw2 = jnp.dot(h, w2_ref[...], preferred_element_type=jnp.float32)
    y2_ref[...] = xw2 * dis


_tc2 = pl.pallas_call(
    _tc2_body,
    out_shape=jax.ShapeDtypeStruct((N_PAD, D), jnp.float32),
)


def _tc3_body(q_ref, dis_ref, b2_ref, out_ref):
    out_ref[...] = jnp.maximum(
        (q_ref[0] + q_ref[1]) * dis_ref[...] + b2_ref[...], 0.0
    )


_tc3 = pl.pallas_call(
    _tc3_body,
    out_shape=jax.ShapeDtypeStruct((N_PAD, D), jnp.float32),
)


@jax.jit
def kernel(x, edge_index, edge_attr, W1, b1, W2, b2):
    row = edge_index[0]
    col = edge_index[1]
    pad_e = E_PAD - E_EDGES
    row_p = jnp.concatenate([row, jnp.zeros((pad_e,), jnp.int32)])
    col_p = jnp.concatenate([col, jnp.zeros((pad_e,), jnp.int32)])
    w_p = jnp.concatenate([edge_attr, jnp.zeros((pad_e,), jnp.float32)])
    row_p = row_p.reshape(NW * CPW, CHUNK)
    col_p = col_p.reshape(NW * CPW, CHUNK)
    w_p = w_p.reshape(NW * CPW, CHUNK)

    x_p = jnp.concatenate(
        [x, jnp.zeros((N_PAD - N_NODES, x.shape[1]), jnp.float32)]
    )
    zero1 = jnp.zeros((N_PAD,), jnp.float32)
    zero2 = jnp.zeros((N_PAD, D), jnp.float32)

    pdeg = _sc_degree(col_p, w_p, zero1)
    dis, y1 = _tc1(pdeg.reshape(2, N_PAD, 1), x_p, W1)

    p1 = _sc_edge(y1, row_p, col_p, w_p, zero2)
    y2 = _tc2(p1, dis, b1.reshape(1, D), W2)

    p2 = _sc_edge(y2, row_p, col_p, w_p, zero2)
    out = _tc3(p2, dis, b2.reshape(1, D))
    return out[:N_NODES]
